# baseline (speedup 1.0000x reference)
import functools

import jax
import jax.numpy as jnp
from jax import lax
from jax.experimental import pallas as pl
from jax.experimental.pallas import tpu as pltpu
from jax.experimental.pallas import tpu_sc as plsc

_F = 26
_V = 100000
_D = 64
_DH = _D // 2
_B = 4096

_NC = 2
_NS = 16
_NW = _NC * _NS
_ROWS = _B * _F
_RPW = _ROWS // _NW

_VP = 100096
_TABH = _F * _DH * _VP

_K0_TR = _F * _DH // 8
_K0_VW = 4864
_K0_NVW = 20
_K0_TAIL0 = _K0_NVW * _K0_VW
_K0_TAILW = 2688
_K0_TB0 = _K0_TAIL0 + _K0_TAILW
_K0_TBW = _V - _K0_TB0

_CR = 128
_NCH = _RPW // _CR

_BB = 512


def _sc_depad(tabT):
    mesh = plsc.VectorSubcoreMesh(core_axis_name="c", subcore_axis_name="s")

    @functools.partial(
        pl.kernel,
        mesh=mesh,
        out_type=jax.ShapeDtypeStruct((_TABH,), jnp.float32),
        scratch_types=[
            pltpu.VMEM((8, _K0_VW), jnp.float32),
            pltpu.VMEM((8, _K0_VW), jnp.float32),
            pltpu.VMEM((8, _K0_TBW), jnp.float32),
            pltpu.VMEM((8, _K0_TBW), jnp.float32),
            pltpu.SemaphoreType.DMA,
            pltpu.SemaphoreType.DMA,
            pltpu.SemaphoreType.DMA,
            pltpu.SemaphoreType.DMA,
        ],
        compiler_params=pltpu.CompilerParams(
            use_tc_tiling_on_sc=True, needs_layout_passes=False),
    )
    def k(tab_hbm, out_hbm, buf0, buf1, tb0, tb1, gs0, gs1, os0, os1):
        wid = lax.axis_index("s") * _NC + lax.axis_index("c")
        gsems = (gs0, gs1)
        osems = (os0, os1)

        def run_pass(unit_of_it, nit, v0_of, vw, bufs):
            def coords(it):
                tr, cv = unit_of_it(it)
                f = tr // (_DH // 8)
                d0 = pl.multiple_of((tr % (_DH // 8)) * 8, 8)
                v0 = v0_of(cv)
                return f, d0, v0, tr

            def issue(it, b):
                f, d0, v0, _ = coords(it)
                pltpu.async_copy(
                    tab_hbm.at[f, pl.ds(d0, 8), pl.ds(v0, vw)],
                    bufs[b].at[:, pl.ds(0, vw)], gsems[b])

            def wait_in(it, b):
                f, d0, v0, _ = coords(it)
                pltpu.make_async_copy(
                    tab_hbm.at[f, pl.ds(d0, 8), pl.ds(v0, vw)],
                    bufs[b].at[:, pl.ds(0, vw)], gsems[b]).wait()

            def drain_out(b):
                for dd in range(8):
                    pltpu.make_async_copy(
                        bufs[b].at[dd, pl.ds(0, vw)],
                        out_hbm.at[pl.ds(0, vw)], osems[b]).wait()

            issue(0, 0)

            def step(i, _):
                for b in range(2):
                    it = 2 * i + b
                    nb = (b + 1) % 2

                    @pl.when(it + 1 < nit)
                    def _():
                        issue(it + 1, nb)

                    f, d0, v0, tr = coords(it)
                    wait_in(it, b)

                    @pl.when(it >= 2)
                    def _():
                        drain_out(b)

                    rbase = ((tr // (_DH // 8)) * _DH
                             + (tr % (_DH // 8)) * 8) * _VP + v0
                    for dd in range(8):
                        pltpu.async_copy(
                            bufs[b].at[dd, pl.ds(0, vw)],
                            out_hbm.at[pl.ds(
                                pl.multiple_of(rbase + dd * _VP, 8), vw)],
                            osems[b])
                return 0

            lax.fori_loop(0, nit // 2, step, 0)
            for b in range(2):
                drain_out(b)

        n1 = _K0_TR * _K0_NVW
        npw = n1 // _NW
        run_pass(
            lambda it: (jnp.minimum(wid * npw + it, n1 - 1) // _K0_NVW,
                        jnp.minimum(wid * npw + it, n1 - 1) % _K0_NVW),
            npw + 1,
            lambda cv: cv * _K0_VW,
            _K0_VW, (buf0, buf1))
        tail_units = lambda it: (jnp.minimum(it * _NW + wid, _K0_TR - 1), 0)
        run_pass(tail_units, 4, lambda cv: _K0_TAIL0, _K0_TAILW, (buf0, buf1))
        run_pass(tail_units, 4, lambda cv: _K0_TB0, _K0_TBW, (tb0, tb1))

    return k(tabT)


def _tc_depad(tabT):
    def body(in_ref, out_ref):
        for k in range(_DH):
            out_ref[pl.ds(k * _VP, _V)] = in_ref[0, k, :]

    return pl.pallas_call(
        body,
        grid=(_F,),
        in_specs=[pl.BlockSpec((1, _DH, _V), lambda f: (f, 1, 0))],
        out_specs=pl.BlockSpec((_DH * _VP,), lambda f: (f,)),
        out_shape=jax.ShapeDtypeStruct((_TABH,), jnp.float32),
        compiler_params=pltpu.CompilerParams(
            dimension_semantics=("arbitrary",),
            vmem_limit_bytes=63 * 1024 * 1024,
        ),
    )(tabT)


def _sc_gather(tab_lo, tab_hi, base_idx):
    mesh = plsc.VectorSubcoreMesh(core_axis_name="c", subcore_axis_name="s")

    @functools.partial(
        pl.kernel,
        mesh=mesh,
        out_type=(jax.ShapeDtypeStruct((_ROWS * _DH,), jnp.float32),
                  jax.ShapeDtypeStruct((_ROWS * _DH,), jnp.float32)),
        scratch_types=[
            pltpu.VMEM((_RPW,), jnp.int32),
            pltpu.VMEM((_CR * _DH,), jnp.int32),
            pltpu.VMEM((_CR * _DH,), jnp.int32),
            pltpu.VMEM((_CR * _DH,), jnp.float32),
            pltpu.VMEM((_CR * _DH,), jnp.float32),
            pltpu.VMEM((_CR * _DH,), jnp.float32),
            pltpu.VMEM((_CR * _DH,), jnp.float32),
            pltpu.SemaphoreType.DMA,
            pltpu.SemaphoreType.DMA,
            pltpu.SemaphoreType.DMA,
            pltpu.SemaphoreType.DMA,
        ],
        compiler_params=pltpu.CompilerParams(
            use_tc_tiling_on_sc=False, needs_layout_passes=False),
    )
    def k(lo_hbm, hi_hbm, base_hbm, olo_hbm, ohi_hbm, base_v, ei0, ei1,
          lo0, lo1, hi0, hi1, gs0, gs1, os0, os1):
        wid = lax.axis_index("s") * _NC + lax.axis_index("c")
        rbase = wid * _RPW
        pltpu.sync_copy(base_hbm.at[pl.ds(rbase, _RPW)], base_v)
        eis = (ei0, ei1)
        los = (lo0, lo1)
        his = (hi0, hi1)
        gsems = (gs0, gs1)
        osems = (os0, os1)

        def gen_idx(c, ei):
            for g in range(_CR // 16):
                j16 = lax.iota(jnp.int32, 16) + (16 * g)
                b16 = base_v[pl.ds(c * _CR + 16 * g, 16)]

                def body_d(d4, _):
                    for u in range(4):
                        d = d4 * 4 + u
                        plsc.store_scatter(
                            ei, [j16 * _DH + d], b16 + d * _VP)
                    return 0

                lax.fori_loop(0, _DH // 4, body_d, 0)

        def issue(c, b):
            pltpu.async_copy(lo_hbm.at[eis[b]], los[b], gsems[b])
            pltpu.async_copy(hi_hbm.at[eis[b]], his[b], gsems[b])

        def wait_in(b):
            pltpu.make_async_copy(lo_hbm.at[eis[b]], los[b], gsems[b]).wait()
            pltpu.make_async_copy(hi_hbm.at[eis[b]], his[b], gsems[b]).wait()

        def drain_out(b):
            pltpu.make_async_copy(
                los[b], olo_hbm.at[pl.ds(0, _CR * _DH)], osems[b]).wait()
            pltpu.make_async_copy(
                his[b], ohi_hbm.at[pl.ds(0, _CR * _DH)], osems[b]).wait()

        gen_idx(0, ei0)
        issue(0, 0)

        def step(i, _):
            for b in range(2):
                c = 2 * i + b
                nb = (b + 1) % 2

                @pl.when(c + 1 < _NCH)
                def _():
                    gen_idx(c + 1, eis[nb])
                    issue(c + 1, nb)

                wait_in(b)

                @pl.when(c >= 2)
                def _():
                    drain_out(b)

                o0 = (rbase + c * _CR) * _DH
                pltpu.async_copy(
                    los[b], olo_hbm.at[pl.ds(o0, _CR * _DH)], osems[b])
                pltpu.async_copy(
                    his[b], ohi_hbm.at[pl.ds(o0, _CR * _DH)], osems[b])
            return 0

        lax.fori_loop(0, _NCH // 2, step, 0)
        for b in range(2):
            drain_out(b)

    return k(tab_lo, tab_hi, base_idx)


def _tc_body(dfp, sx, dW0, db0, dW1, db1, dW2, db2,
             cW0, cb0, cW1, cb1, cW2, cb2,
             tW0d, tW0x, tb0, tW1, tb1, tW2, tb2, tW3, tb3,
             fW, fb, out):
    f32 = jnp.float32
    bf = jnp.bfloat16
    mm = lambda a, w: jnp.dot(a.astype(bf), w[...],
                              preferred_element_type=f32)
    x = jnp.maximum(mm(dfp[...], dW0) + db0[...], 0.0)
    x = jnp.maximum(mm(x, dW1) + db1[...], 0.0)
    dxp = jnp.maximum(mm(x, dW2) + db2[...], 0.0)
    x0 = jnp.concatenate([dxp, sx[...]], axis=1)
    xi = x0
    for cW, cb in ((cW0, cb0), (cW1, cb1), (cW2, cb2)):
        xi = x0 * (mm(xi, cW) + cb[...]) + xi
    h = mm(dxp, tW0d) + mm(xi, tW0x) + tb0[...]
    h = jnp.maximum(h, 0.0)
    h = jnp.maximum(mm(h, tW1) + tb1[...], 0.0)
    h = jnp.maximum(mm(h, tW2) + tb2[...], 0.0)
    h = jnp.maximum(mm(h, tW3) + tb3[...], 0.0)
    z = mm(h, fW) + fb[...]
    out[...] = 1.0 / (1.0 + jnp.exp(-z))


def _dense_stack(dfp, sx, weights):
    nb = _B // _BB
    full = lambda s: pl.BlockSpec(s, lambda i: (0, 0))
    in_specs = [
        pl.BlockSpec((_BB, 128), lambda i: (i, 0)),
        pl.BlockSpec((_BB, _F * _D), lambda i: (i, 0)),
    ] + [full(w.shape) for w in weights]
    return pl.pallas_call(
        _tc_body,
        grid=(nb,),
        in_specs=in_specs,
        out_specs=pl.BlockSpec((_BB, 1), lambda i: (i, 0)),
        out_shape=jax.ShapeDtypeStruct((_B, 1), jnp.float32),
        compiler_params=pltpu.CompilerParams(
            dimension_semantics=("arbitrary",),
            vmem_limit_bytes=63 * 1024 * 1024,
        ),
    )(dfp, sx, *weights)


def kernel(dense_features, sparse_features, tables, dense_Ws, dense_bs,
           cross_Ws, cross_bs, top_Ws, top_bs, final_W, final_b):
    f32 = jnp.float32
    tabT = jnp.transpose(tables, (0, 2, 1))
    tab_lo = _sc_depad(tabT)
    tab_hi = _tc_depad(tabT)
    base_idx = (sparse_features.astype(jnp.int32)
                + (jnp.arange(_F, dtype=jnp.int32) * (_DH * _VP))[None, :]
                ).reshape(-1)
    olo, ohi = _sc_gather(tab_lo, tab_hi, base_idx)
    sx = jnp.concatenate(
        [olo.reshape(_B, _F, _DH), ohi.reshape(_B, _F, _DH)],
        axis=2).reshape(_B, _F * _D)

    z = lambda r, c: jnp.zeros((r, c), f32)
    dfp = jnp.pad(dense_features, ((0, 0), (0, 128 - dense_features.shape[1])))
    dW0 = jnp.pad(dense_Ws[0], ((0, 128 - dense_Ws[0].shape[0]), (0, 0)))
    db0 = dense_bs[0].reshape(1, -1)
    dW1 = dense_Ws[1]
    db1 = dense_bs[1].reshape(1, -1)
    dW2 = jnp.pad(dense_Ws[2], ((0, 0), (0, 64)))
    db2 = jnp.pad(dense_bs[2], ((0, 64),)).reshape(1, 128)

    cWs, cbs = [], []
    for W, b in zip(cross_Ws, cross_bs):
        t = jnp.concatenate([W[:64], z(64, W.shape[1]), W[64:]], axis=0)
        Wp = jnp.concatenate([t[:, :64], z(t.shape[0], 64), t[:, 64:]], axis=1)
        cWs.append(Wp)
        cbs.append(jnp.concatenate(
            [b[:64], jnp.zeros((64,), f32), b[64:]]).reshape(1, -1))

    tW0d = jnp.concatenate([top_Ws[0][:64], z(64, 1024)], axis=0)
    tW0x = jnp.concatenate(
        [top_Ws[0][64:128], z(64, 1024), top_Ws[0][128:]], axis=0)
    tb0 = top_bs[0].reshape(1, -1)
    tW1, tb1 = top_Ws[1], top_bs[1].reshape(1, -1)
    tW2, tb2 = top_Ws[2], top_bs[2].reshape(1, -1)
    tW3, tb3 = top_Ws[3], top_bs[3].reshape(1, -1)
    fb = final_b.reshape(1, 1)

    bf = jnp.bfloat16
    weights = [dW0.astype(bf), db0, dW1.astype(bf), db1, dW2.astype(bf), db2,
               cWs[0].astype(bf), cbs[0], cWs[1].astype(bf), cbs[1],
               cWs[2].astype(bf), cbs[2],
               tW0d.astype(bf), tW0x.astype(bf), tb0,
               tW1.astype(bf), tb1, tW2.astype(bf), tb2, tW3.astype(bf), tb3,
               final_W.astype(bf), fb]
    return _dense_stack(dfp, sx, weights)
